# baseline (speedup 1.0000x reference)
import functools

import jax
import jax.numpy as jnp
from jax.experimental import pallas as pl
from jax.experimental.pallas import tpu as pltpu

N = 10000
E = 160000
IN_FEAT = 256
HID = 256
C = 2
HC = HID * C
OUT = 40
BETA = 0.5
EPS = 1e-5


def _ln(x, g, b):
    mu = x.mean(axis=-1, keepdims=True)
    var = ((x - mu) ** 2).mean(axis=-1, keepdims=True)
    return (x - mu) / jnp.sqrt(var + EPS) * g + b


def _dense0_body(x_ref, w_ref, b_ref, g_ref, bb_ref, o_ref):
    h = jnp.dot(x_ref[...], w_ref[...], preferred_element_type=jnp.float32)
    h = jax.nn.relu(h + b_ref[...])
    o_ref[...] = _ln(h, g_ref[...], bb_ref[...])


def _dense0(x, W1, b1, g, bb):
    BM = 1000
    return pl.pallas_call(
        _dense0_body,
        grid=(N // BM,),
        in_specs=[
            pl.BlockSpec((BM, IN_FEAT), lambda i: (i, 0)),
            pl.BlockSpec((IN_FEAT, HC), lambda i: (0, 0)),
            pl.BlockSpec((HC,), lambda i: (0,)),
            pl.BlockSpec((HC,), lambda i: (0,)),
            pl.BlockSpec((HC,), lambda i: (0,)),
        ],
        out_specs=pl.BlockSpec((BM, HC), lambda i: (i, 0)),
        out_shape=jax.ShapeDtypeStruct((N, HC), jnp.float32),
    )(x, W1, b1, g, bb)


def _mm_body(a_ref, w_ref, o_ref):
    o_ref[...] = jnp.dot(a_ref[...], w_ref[...], preferred_element_type=jnp.float32)


def _mm(a, w):
    BM = 1000
    m, k = a.shape
    n = w.shape[1]
    return pl.pallas_call(
        _mm_body,
        grid=(m // BM,),
        in_specs=[
            pl.BlockSpec((BM, k), lambda i: (i, 0)),
            pl.BlockSpec((k, n), lambda i: (0, 0)),
        ],
        out_specs=pl.BlockSpec((BM, n), lambda i: (i, 0)),
        out_shape=jax.ShapeDtypeStruct((m, n), jnp.float32),
    )(a, w)


def _post_body(seg_ref, ego_ref, g_ref, b_ref, o_ref):
    h2 = _ln(jax.nn.relu(seg_ref[...]), g_ref[...], b_ref[...])
    o_ref[...] = (1.0 - BETA) * h2 + BETA * ego_ref[...]


def _post(seg, ego, g, b):
    BM = 1000
    return pl.pallas_call(
        _post_body,
        grid=(N // BM,),
        in_specs=[
            pl.BlockSpec((BM, HC), lambda i: (i, 0)),
            pl.BlockSpec((BM, HC), lambda i: (i, 0)),
            pl.BlockSpec((HC,), lambda i: (0,)),
            pl.BlockSpec((HC,), lambda i: (0,)),
        ],
        out_specs=pl.BlockSpec((BM, HC), lambda i: (i, 0)),
        out_shape=jax.ShapeDtypeStruct((N, HC), jnp.float32),
    )(seg, ego, g, b)


def _final_body(h_ref, w_ref, b_ref, o_ref):
    o = jnp.dot(h_ref[...], w_ref[...], preferred_element_type=jnp.float32) + b_ref[...]
    o_ref[...] = jax.nn.log_softmax(o, axis=-1)


def _final(h, W2, b2):
    BM = 1000
    return pl.pallas_call(
        _final_body,
        grid=(N // BM,),
        in_specs=[
            pl.BlockSpec((BM, HC), lambda i: (i, 0)),
            pl.BlockSpec((HC, OUT), lambda i: (0, 0)),
            pl.BlockSpec((OUT,), lambda i: (0,)),
        ],
        out_specs=pl.BlockSpec((BM, OUT), lambda i: (i, 0)),
        out_shape=jax.ShapeDtypeStruct((N, OUT), jnp.float32),
    )(h, W2, b2)


def _edge_layer(h, row, col, w_lin, w_att):
    hp = _mm(h, w_lin)
    bin_rela = jax.nn.relu(0.5 * hp[row] + hp[col])
    bin_rela = bin_rela @ w_att
    bin_rela = jax.nn.softmax(bin_rela, axis=1)
    h_col = hp[col]
    x_j = jnp.concatenate([h_col * bin_rela[:, i:i + 1] for i in range(C)], axis=1)
    return jax.ops.segment_sum(x_j, row, num_segments=N)


def kernel(x, edge_index, W1, b1, ln0_g, ln0_b, lin_w0, att_w0, ln1_g, ln1_b,
           lin_w1, att_w1, ln2_g, ln2_b, W2, b2):
    row = edge_index[0]
    col = edge_index[1]
    mask = row != col
    row = jnp.where(mask, row, N)
    h = _dense0(x, W1, b1, ln0_g, ln0_b)
    ego = h
    for (wl, wa, g, bb) in ((lin_w0, att_w0, ln1_g, ln1_b),
                            (lin_w1, att_w1, ln2_g, ln2_b)):
        seg = _edge_layer(h, row, col, wl, wa)
        h = _post(seg, ego, g, bb)
    return _final(h, W2, b2)
